# Initial kernel scaffold; baseline (speedup 1.0000x reference)
import jax
import jax.numpy as jnp
from jax.experimental import pallas as pl


def kernel(src_features, edge_sh, edge_emb, edge_index, W1, b1, W2, b2):
    raise NotImplementedError("write your pallas kernel here")


# speedup vs baseline: 3.6772x; 3.6772x over previous
import functools

import jax
import jax.numpy as jnp
from jax import lax
from jax.experimental import pallas as pl
from jax.experimental.pallas import tpu as pltpu
from jax.experimental.pallas import tpu_sc as plsc

_NC, _NS = 2, 16
_NW = _NC * _NS
_C = 100
_G = 10


def _gather_call(sf, idx3, E, N, F):
    EW = E // _NW
    NCH = EW // _C
    NG = NCH // _G
    mesh = plsc.VectorSubcoreMesh(core_axis_name="c", subcore_axis_name="s")

    @functools.partial(
        pl.kernel,
        out_type=jax.ShapeDtypeStruct((E, F), jnp.float32),
        mesh=mesh,
        compiler_params=pltpu.CompilerParams(use_tc_tiling_on_sc=False),
        scratch_types=[
            pltpu.VMEM((NCH, _C), jnp.int32),
            pltpu.VMEM((_G * _C, F), jnp.float32),
            pltpu.SemaphoreType.DMA,
        ],
    )
    def k(sf_hbm, idx_hbm, out_hbm, idx_v, rows_v, sem):
        cid = lax.axis_index("c")
        sid = lax.axis_index("s")
        wid = sid * _NC + cid
        pltpu.sync_copy(idx_hbm.at[wid], idx_v)

        def body(g, carry):
            descs = []
            for j in range(_G):
                d = pltpu.async_copy(
                    sf_hbm.at[idx_v.at[g * _G + j]],
                    rows_v.at[pl.ds(j * _C, _C)],
                    sem,
                )
                descs.append(d)
            for d in descs:
                d.wait()
            pltpu.sync_copy(
                rows_v, out_hbm.at[pl.ds(wid * EW + g * (_G * _C), _G * _C)]
            )
            return carry

        lax.fori_loop(0, NG, body, 0)

    return k(sf, idx3)


def _msg_call(xg, sh, emb, W1T, b1r, W2n_s, T2, S2, b2m_s, E, F):
    BE = 1600
    grid = E // BE
    inv_sqrt2 = 0.7071067811865476

    def body(xg_ref, sh_ref, emb_ref, w1t_ref, b1_ref, w2n_ref, t2_ref,
             s2_ref, b2m_ref, out_ref):
        h0 = jnp.dot(emb_ref[...], w1t_ref[...],
                     preferred_element_type=jnp.float32) + b1_ref[...]
        h = 0.5 * h0 * (1.0 + lax.erf(h0 * inv_sqrt2))
        x = xg_ref[...] * sh_ref[...]
        a2 = jnp.dot(x, w2n_ref[...], preferred_element_type=jnp.float32)
        ht = jnp.dot(h, t2_ref[...], preferred_element_type=jnp.float32)
        msg = jnp.dot(ht * a2, s2_ref[...],
                      preferred_element_type=jnp.float32)
        msg = msg + jnp.dot(x, b2m_ref[...],
                            preferred_element_type=jnp.float32)
        out_ref[...] = msg

    wspec = lambda shape: pl.BlockSpec(shape, lambda i: (0,) * len(shape))
    return pl.pallas_call(
        body,
        grid=(grid,),
        in_specs=[
            pl.BlockSpec((BE, F), lambda i: (i, 0)),
            pl.BlockSpec((BE, 1), lambda i: (i, 0)),
            pl.BlockSpec((BE, F), lambda i: (i, 0)),
            wspec((F, F)),
            wspec((1, F)),
            wspec((F, F * F)),
            wspec((F, F * F)),
            wspec((F * F, F)),
            wspec((F, F)),
        ],
        out_specs=pl.BlockSpec((BE, F), lambda i: (i, 0)),
        out_shape=jax.ShapeDtypeStruct((E, F), jnp.float32),
    )(xg, sh, emb, W1T, b1r, W2n_s, T2, S2, b2m_s)


def _scatter_call(msg, idx3, E, N, F):
    EW = E // _NW
    NCH = EW // _C
    NG = NCH // _G
    RPT = N // _NS
    mesh = plsc.VectorSubcoreMesh(core_axis_name="c", subcore_axis_name="s")

    @functools.partial(
        pl.kernel,
        out_type=(
            jax.ShapeDtypeStruct((_NC, N, F), jnp.float32),
            jax.ShapeDtypeStruct((_NC, N, F), jnp.float32),
        ),
        mesh=mesh,
        compiler_params=pltpu.CompilerParams(use_tc_tiling_on_sc=False),
        scratch_types=[
            pltpu.VMEM((NCH, _C), jnp.int32),
            pltpu.VMEM((_G * _C, F), jnp.float32),
            pltpu.VMEM((_C, F), jnp.float32),
            pltpu.VMEM((RPT, F), jnp.float32),
            pltpu.VMEM_SHARED((N, F), jnp.float32),
            pltpu.VMEM_SHARED((N, F), jnp.float32),
            pltpu.SemaphoreType.DMA,
            pltpu.SemaphoreType.DMA,
        ],
    )
    def k(msg_hbm, idx_hbm, pmsg_hbm, pdeg_hbm, idx_v, msg_v, ones_v,
          zero_v, accm_sh, accd_sh, sem_m, sem_d):
        cid = lax.axis_index("c")
        sid = lax.axis_index("s")
        wid = sid * _NC + cid

        def initz(r, carry):
            zero_v[r, :] = jnp.zeros((F,), jnp.float32)
            return carry

        lax.fori_loop(0, RPT, initz, 0)

        def inito(r, carry):
            ones_v[r, :] = jnp.ones((F,), jnp.float32)
            return carry

        lax.fori_loop(0, _C, inito, 0)

        pltpu.sync_copy(zero_v, accm_sh.at[pl.ds(sid * RPT, RPT)])
        pltpu.sync_copy(zero_v, accd_sh.at[pl.ds(sid * RPT, RPT)])
        plsc.subcore_barrier()

        pltpu.sync_copy(idx_hbm.at[wid], idx_v)

        def body(g, carry):
            pltpu.sync_copy(
                msg_hbm.at[pl.ds(wid * EW + g * (_G * _C), _G * _C)], msg_v
            )
            descs = []
            for j in range(_G):
                descs.append(pltpu.async_copy(
                    msg_v.at[pl.ds(j * _C, _C)],
                    accm_sh.at[idx_v.at[g * _G + j]],
                    sem_m, add=True,
                ))
                descs.append(pltpu.async_copy(
                    ones_v,
                    accd_sh.at[idx_v.at[g * _G + j]],
                    sem_d, add=True,
                ))
            for d in descs:
                d.wait()
            return carry

        lax.fori_loop(0, NG, body, 0)
        plsc.subcore_barrier()

        pltpu.sync_copy(accm_sh.at[pl.ds(sid * RPT, RPT)],
                        pmsg_hbm.at[cid, pl.ds(sid * RPT, RPT)])
        pltpu.sync_copy(accd_sh.at[pl.ds(sid * RPT, RPT)],
                        pdeg_hbm.at[cid, pl.ds(sid * RPT, RPT)])

    return k(msg, idx3)


def _combine_call(pmsg, pdeg, N, F):
    def body(pm_ref, pd_ref, out_ref):
        s = pm_ref[0] + pm_ref[1]
        d = pd_ref[0] + pd_ref[1]
        out_ref[...] = s / jnp.maximum(d, 1.0)

    return pl.pallas_call(
        body,
        out_shape=jax.ShapeDtypeStruct((N, F), jnp.float32),
    )(pmsg, pdeg)


def kernel(src_features, edge_sh, edge_emb, edge_index, W1, b1, W2, b2):
    N, F = src_features.shape
    E = edge_emb.shape[0]
    EW = E // _NW
    NCH = EW // _C

    src3 = edge_index[0].reshape(_NW, NCH, _C)
    dst3 = edge_index[1].reshape(_NW, NCH, _C)

    scale = 0.25
    W1T = W1.T
    b1r = b1.reshape(1, F)
    W2n_s = (W2.reshape(F, F, F).reshape(F, F * F)) * scale
    T2 = jnp.tile(jnp.eye(F, dtype=jnp.float32), (1, F))
    S2 = jnp.kron(jnp.eye(F, dtype=jnp.float32),
                  jnp.ones((F, 1), dtype=jnp.float32))
    b2m_s = b2.reshape(F, F) * scale

    xg = _gather_call(src_features, src3, E, N, F)
    msg = _msg_call(xg, edge_sh, edge_emb, W1T, b1r, W2n_s, T2, S2, b2m_s,
                    E, F)
    pmsg, pdeg = _scatter_call(msg, dst3, E, N, F)
    return _combine_call(pmsg, pdeg, N, F)


# speedup vs baseline: 4.2021x; 1.1428x over previous
import functools

import jax
import jax.numpy as jnp
from jax import lax
from jax.experimental import pallas as pl
from jax.experimental.pallas import tpu as pltpu
from jax.experimental.pallas import tpu_sc as plsc

_NC, _NS = 2, 16
_NW = _NC * _NS
_C = 100
_G = 10


def _gather_call(sf, idx3, E, N, F):
    EW = E // _NW
    NCH = EW // _C
    NG = NCH // _G
    mesh = plsc.VectorSubcoreMesh(core_axis_name="c", subcore_axis_name="s")

    @functools.partial(
        pl.kernel,
        out_type=jax.ShapeDtypeStruct((E, F), jnp.bfloat16),
        mesh=mesh,
        compiler_params=pltpu.CompilerParams(use_tc_tiling_on_sc=False),
        scratch_types=[
            pltpu.VMEM((NCH, _C), jnp.int32),
            pltpu.VMEM((_G * _C, F), jnp.bfloat16),
            pltpu.SemaphoreType.DMA,
        ],
    )
    def k(sf_hbm, idx_hbm, out_hbm, idx_v, rows_v, sem):
        cid = lax.axis_index("c")
        sid = lax.axis_index("s")
        wid = sid * _NC + cid
        pltpu.sync_copy(idx_hbm.at[wid], idx_v)

        def body(g, carry):
            descs = []
            for j in range(_G):
                d = pltpu.async_copy(
                    sf_hbm.at[idx_v.at[g * _G + j]],
                    rows_v.at[pl.ds(j * _C, _C)],
                    sem,
                )
                descs.append(d)
            for d in descs:
                d.wait()
            pltpu.sync_copy(
                rows_v, out_hbm.at[pl.ds(wid * EW + g * (_G * _C), _G * _C)]
            )
            return carry

        lax.fori_loop(0, NG, body, 0)

    return k(sf, idx3)


def _msg_call(xg, sh, emb, W1T, b1r, W2n_s, T2, S2, b2m_s, E, F):
    BE = 6400
    grid = E // BE
    inv_sqrt2 = 0.7071067811865476

    def body(xg_ref, sh_ref, emb_ref, w1t_ref, b1_ref, w2n_ref, t2_ref,
             s2_ref, b2m_ref, out_ref):
        emb16 = emb_ref[...].astype(jnp.bfloat16)
        h0 = jnp.dot(emb16, w1t_ref[...],
                     preferred_element_type=jnp.float32) + b1_ref[...]
        h = 0.5 * h0 * (1.0 + lax.erf(h0 * inv_sqrt2))
        h16 = h.astype(jnp.bfloat16)
        ht = jnp.dot(h16, t2_ref[...], preferred_element_type=jnp.float32)
        x16 = (xg_ref[...] * sh_ref[...]).astype(jnp.bfloat16)
        a2 = jnp.dot(x16, w2n_ref[...], preferred_element_type=jnp.float32)
        p16 = (ht * a2).astype(jnp.bfloat16)
        msg = jnp.dot(p16, s2_ref[...], preferred_element_type=jnp.float32)
        msg = msg + jnp.dot(x16, b2m_ref[...],
                            preferred_element_type=jnp.float32)
        out_ref[...] = msg

    wspec = lambda shape: pl.BlockSpec(shape, lambda i: (0,) * len(shape))
    return pl.pallas_call(
        body,
        grid=(grid,),
        in_specs=[
            pl.BlockSpec((BE, F), lambda i: (i, 0)),
            pl.BlockSpec((BE, 1), lambda i: (i, 0)),
            pl.BlockSpec((BE, F), lambda i: (i, 0)),
            wspec((F, F)),
            wspec((1, F)),
            wspec((F, F * F)),
            wspec((F, F * F)),
            wspec((F * F, F)),
            wspec((F, F)),
        ],
        out_specs=pl.BlockSpec((BE, F), lambda i: (i, 0)),
        out_shape=jax.ShapeDtypeStruct((E, F), jnp.float32),
    )(xg, sh, emb, W1T, b1r, W2n_s, T2, S2, b2m_s)


def _scatter_call(msg, idx3, E, N, F):
    EW = E // _NW
    NCH = EW // _C
    NG = NCH // _G
    RPT = N // _NS
    mesh = plsc.VectorSubcoreMesh(core_axis_name="c", subcore_axis_name="s")

    @functools.partial(
        pl.kernel,
        out_type=(
            jax.ShapeDtypeStruct((_NC, N, F), jnp.float32),
            jax.ShapeDtypeStruct((_NC, N, F), jnp.float32),
        ),
        mesh=mesh,
        compiler_params=pltpu.CompilerParams(use_tc_tiling_on_sc=False),
        scratch_types=[
            pltpu.VMEM((NCH, _C), jnp.int32),
            pltpu.VMEM((_G * _C, F), jnp.float32),
            pltpu.VMEM((_C, F), jnp.float32),
            pltpu.VMEM((RPT, F), jnp.float32),
            pltpu.VMEM_SHARED((N, F), jnp.float32),
            pltpu.VMEM_SHARED((N, F), jnp.float32),
            pltpu.SemaphoreType.DMA,
            pltpu.SemaphoreType.DMA,
        ],
    )
    def k(msg_hbm, idx_hbm, pmsg_hbm, pdeg_hbm, idx_v, msg_v, ones_v,
          zero_v, accm_sh, accd_sh, sem_m, sem_d):
        cid = lax.axis_index("c")
        sid = lax.axis_index("s")
        wid = sid * _NC + cid

        def initz(r, carry):
            zero_v[r, :] = jnp.zeros((F,), jnp.float32)
            return carry

        lax.fori_loop(0, RPT, initz, 0)

        def inito(r, carry):
            ones_v[r, :] = jnp.ones((F,), jnp.float32)
            return carry

        lax.fori_loop(0, _C, inito, 0)

        pltpu.sync_copy(zero_v, accm_sh.at[pl.ds(sid * RPT, RPT)])
        pltpu.sync_copy(zero_v, accd_sh.at[pl.ds(sid * RPT, RPT)])
        plsc.subcore_barrier()

        pltpu.sync_copy(idx_hbm.at[wid], idx_v)

        def body(g, carry):
            pltpu.sync_copy(
                msg_hbm.at[pl.ds(wid * EW + g * (_G * _C), _G * _C)], msg_v
            )
            descs = []
            for j in range(_G):
                descs.append(pltpu.async_copy(
                    msg_v.at[pl.ds(j * _C, _C)],
                    accm_sh.at[idx_v.at[g * _G + j]],
                    sem_m, add=True,
                ))
                descs.append(pltpu.async_copy(
                    ones_v,
                    accd_sh.at[idx_v.at[g * _G + j]],
                    sem_d, add=True,
                ))
            for d in descs:
                d.wait()
            return carry

        lax.fori_loop(0, NG, body, 0)
        plsc.subcore_barrier()

        pltpu.sync_copy(accm_sh.at[pl.ds(sid * RPT, RPT)],
                        pmsg_hbm.at[cid, pl.ds(sid * RPT, RPT)])
        pltpu.sync_copy(accd_sh.at[pl.ds(sid * RPT, RPT)],
                        pdeg_hbm.at[cid, pl.ds(sid * RPT, RPT)])

    return k(msg, idx3)


def _combine_call(pmsg, pdeg, N, F):
    def body(pm_ref, pd_ref, out_ref):
        s = pm_ref[0] + pm_ref[1]
        d = pd_ref[0] + pd_ref[1]
        out_ref[...] = s / jnp.maximum(d, 1.0)

    return pl.pallas_call(
        body,
        out_shape=jax.ShapeDtypeStruct((N, F), jnp.float32),
    )(pmsg, pdeg)


def kernel(src_features, edge_sh, edge_emb, edge_index, W1, b1, W2, b2):
    N, F = src_features.shape
    E = edge_emb.shape[0]
    EW = E // _NW
    NCH = EW // _C

    src3 = edge_index[0].reshape(_NW, NCH, _C)
    dst3 = edge_index[1].reshape(_NW, NCH, _C)

    scale = 0.25
    W1T = W1.T.astype(jnp.bfloat16)
    b1r = b1.reshape(1, F)
    W2n_s = ((W2.reshape(F, F, F).reshape(F, F * F)) * scale
             ).astype(jnp.bfloat16)
    T2 = jnp.tile(jnp.eye(F, dtype=jnp.bfloat16), (1, F))
    S2 = jnp.kron(jnp.eye(F, dtype=jnp.bfloat16),
                  jnp.ones((F, 1), dtype=jnp.bfloat16))
    b2m_s = (b2.reshape(F, F) * scale).astype(jnp.bfloat16)

    xg = _gather_call(src_features.astype(jnp.bfloat16), src3, E, N, F)
    msg = _msg_call(xg, edge_sh, edge_emb, W1T, b1r, W2n_s, T2, S2, b2m_s,
                    E, F)
    pmsg, pdeg = _scatter_call(msg, dst3, E, N, F)
    return _combine_call(pmsg, pdeg, N, F)


# speedup vs baseline: 4.2548x; 1.0125x over previous
import functools

import jax
import jax.numpy as jnp
from jax import lax
from jax.experimental import pallas as pl
from jax.experimental.pallas import tpu as pltpu
from jax.experimental.pallas import tpu_sc as plsc

_NC, _NS = 2, 16
_NW = _NC * _NS
_C = 100
_G = 10


def _gather_call(sf, idx3, E, N, F):
    EW = E // _NW
    NCH = EW // _C
    NG = NCH // _G
    mesh = plsc.VectorSubcoreMesh(core_axis_name="c", subcore_axis_name="s")

    GC = _G * _C

    @functools.partial(
        pl.kernel,
        out_type=jax.ShapeDtypeStruct((E, F), jnp.float32),
        mesh=mesh,
        compiler_params=pltpu.CompilerParams(use_tc_tiling_on_sc=False),
        scratch_types=[
            pltpu.VMEM((NCH, _C), jnp.int32),
            pltpu.VMEM((2 * GC, F), jnp.float32),
            pltpu.SemaphoreType.DMA,
            pltpu.SemaphoreType.DMA,
            pltpu.SemaphoreType.DMA,
        ],
    )
    def k(sf_hbm, idx_hbm, out_hbm, idx_v, rows_v, gsem, ssem0, ssem1):
        cid = lax.axis_index("c")
        sid = lax.axis_index("s")
        wid = sid * _NC + cid
        pltpu.sync_copy(idx_hbm.at[wid], idx_v)

        def fire(g, boff):
            return [pltpu.async_copy(
                        sf_hbm.at[idx_v.at[g * _G + j]],
                        rows_v.at[pl.ds(boff + j * _C, _C)],
                        gsem)
                    for j in range(_G)]

        def drain_gathers(boff):
            for j in range(_G):
                pltpu.make_async_copy(
                    sf_hbm.at[idx_v.at[j]],
                    rows_v.at[pl.ds(boff + j * _C, _C)],
                    gsem).wait()

        def store(g, boff, sem):
            return pltpu.async_copy(
                rows_v.at[pl.ds(boff, GC)],
                out_hbm.at[pl.ds(wid * EW + g * GC, GC)], sem)

        def drain_store(g, sem):
            pltpu.make_async_copy(
                rows_v.at[pl.ds(0, GC)],
                out_hbm.at[pl.ds(wid * EW + g * GC, GC)], sem).wait()

        fire(0, 0)

        def body(k2, carry):
            g0 = 2 * k2
            drain_gathers(0)

            @pl.when(k2 >= 1)
            def _():
                drain_store(g0 - 1, ssem1)

            d1 = fire(g0 + 1, GC)
            store(g0, 0, ssem0)
            for d in d1:
                d.wait()
            drain_store(g0, ssem0)

            @pl.when(k2 < NG // 2 - 1)
            def _():
                fire(g0 + 2, 0)

            store(g0 + 1, GC, ssem1)
            return carry

        lax.fori_loop(0, NG // 2, body, 0)
        drain_store(NG - 1, ssem1)

    return k(sf, idx3)


def _msg_call(xg, sh, emb, W1T, b1r, W2n_s, T2, S2, b2m_s, E, F):
    BE = 6400
    grid = E // BE
    inv_sqrt2 = 0.7071067811865476

    def body(xg_ref, sh_ref, emb_ref, w1t_ref, b1_ref, w2n_ref, t2_ref,
             s2_ref, b2m_ref, out_ref):
        emb16 = emb_ref[...].astype(jnp.bfloat16)
        h0 = jnp.dot(emb16, w1t_ref[...],
                     preferred_element_type=jnp.float32) + b1_ref[...]
        h = 0.5 * h0 * (1.0 + lax.erf(h0 * inv_sqrt2))
        h16 = h.astype(jnp.bfloat16)
        ht = jnp.dot(h16, t2_ref[...], preferred_element_type=jnp.float32)
        x16 = (xg_ref[...] * sh_ref[...]).astype(jnp.bfloat16)
        a2 = jnp.dot(x16, w2n_ref[...], preferred_element_type=jnp.float32)
        p16 = (ht * a2).astype(jnp.bfloat16)
        msg = jnp.dot(p16, s2_ref[...], preferred_element_type=jnp.float32)
        msg = msg + jnp.dot(x16, b2m_ref[...],
                            preferred_element_type=jnp.float32)
        out_ref[...] = msg

    wspec = lambda shape: pl.BlockSpec(shape, lambda i: (0,) * len(shape))
    return pl.pallas_call(
        body,
        grid=(grid,),
        in_specs=[
            pl.BlockSpec((BE, F), lambda i: (i, 0)),
            pl.BlockSpec((BE, 1), lambda i: (i, 0)),
            pl.BlockSpec((BE, F), lambda i: (i, 0)),
            wspec((F, F)),
            wspec((1, F)),
            wspec((F, F * F)),
            wspec((F, F * F)),
            wspec((F * F, F)),
            wspec((F, F)),
        ],
        out_specs=pl.BlockSpec((BE, F), lambda i: (i, 0)),
        out_shape=jax.ShapeDtypeStruct((E, F), jnp.float32),
    )(xg, sh, emb, W1T, b1r, W2n_s, T2, S2, b2m_s)


def _scatter_call(msg, idx3, E, N, F):
    EW = E // _NW
    NCH = EW // _C
    NG = NCH // _G
    RPT = N // _NS
    mesh = plsc.VectorSubcoreMesh(core_axis_name="c", subcore_axis_name="s")

    @functools.partial(
        pl.kernel,
        out_type=(
            jax.ShapeDtypeStruct((_NC, N, F), jnp.float32),
            jax.ShapeDtypeStruct((_NC, N, F), jnp.float32),
        ),
        mesh=mesh,
        compiler_params=pltpu.CompilerParams(use_tc_tiling_on_sc=False),
        scratch_types=[
            pltpu.VMEM((NCH, _C), jnp.int32),
            pltpu.VMEM((2 * _G * _C, F), jnp.float32),
            pltpu.VMEM((_C, F), jnp.float32),
            pltpu.VMEM((RPT, F), jnp.float32),
            pltpu.VMEM_SHARED((N, F), jnp.float32),
            pltpu.VMEM_SHARED((N, F), jnp.float32),
            pltpu.SemaphoreType.DMA,
            pltpu.SemaphoreType.DMA,
            pltpu.SemaphoreType.DMA,
            pltpu.SemaphoreType.DMA,
        ],
    )
    def k(msg_hbm, idx_hbm, pmsg_hbm, pdeg_hbm, idx_v, msg_v, ones_v,
          zero_v, accm_sh, accd_sh, sem_m, sem_d, lsem0, lsem1):
        cid = lax.axis_index("c")
        sid = lax.axis_index("s")
        wid = sid * _NC + cid
        GC = _G * _C

        def initz(r, carry):
            zero_v[r, :] = jnp.zeros((F,), jnp.float32)
            return carry

        lax.fori_loop(0, RPT, initz, 0)

        def inito(r, carry):
            ones_v[r, :] = jnp.ones((F,), jnp.float32)
            return carry

        lax.fori_loop(0, _C, inito, 0)

        pltpu.sync_copy(zero_v, accm_sh.at[pl.ds(sid * RPT, RPT)])
        pltpu.sync_copy(zero_v, accd_sh.at[pl.ds(sid * RPT, RPT)])
        plsc.subcore_barrier()

        pltpu.sync_copy(idx_hbm.at[wid], idx_v)

        def load(g, boff, sem):
            return pltpu.async_copy(
                msg_hbm.at[pl.ds(wid * EW + g * GC, GC)],
                msg_v.at[pl.ds(boff, GC)], sem)

        def drain_load(g, boff, sem):
            pltpu.make_async_copy(
                msg_hbm.at[pl.ds(wid * EW + g * GC, GC)],
                msg_v.at[pl.ds(boff, GC)], sem).wait()

        def scatter_group(g, boff):
            descs = []
            for j in range(_G):
                descs.append(pltpu.async_copy(
                    msg_v.at[pl.ds(boff + j * _C, _C)],
                    accm_sh.at[idx_v.at[g * _G + j]],
                    sem_m, add=True,
                ))
                descs.append(pltpu.async_copy(
                    ones_v,
                    accd_sh.at[idx_v.at[g * _G + j]],
                    sem_d, add=True,
                ))
            for d in descs:
                d.wait()

        load(0, 0, lsem0)

        def body(k2, carry):
            g0 = 2 * k2
            drain_load(g0, 0, lsem0)
            load(g0 + 1, GC, lsem1)
            scatter_group(g0, 0)

            @pl.when(k2 < NG // 2 - 1)
            def _():
                load(g0 + 2, 0, lsem0)

            drain_load(g0 + 1, GC, lsem1)
            scatter_group(g0 + 1, GC)
            return carry

        lax.fori_loop(0, NG // 2, body, 0)
        plsc.subcore_barrier()

        pltpu.sync_copy(accm_sh.at[pl.ds(sid * RPT, RPT)],
                        pmsg_hbm.at[cid, pl.ds(sid * RPT, RPT)])
        pltpu.sync_copy(accd_sh.at[pl.ds(sid * RPT, RPT)],
                        pdeg_hbm.at[cid, pl.ds(sid * RPT, RPT)])

    return k(msg, idx3)


def _combine_call(pmsg, pdeg, N, F):
    def body(pm_ref, pd_ref, out_ref):
        s = pm_ref[0] + pm_ref[1]
        d = pd_ref[0] + pd_ref[1]
        out_ref[...] = s / jnp.maximum(d, 1.0)

    return pl.pallas_call(
        body,
        out_shape=jax.ShapeDtypeStruct((N, F), jnp.float32),
    )(pmsg, pdeg)


def kernel(src_features, edge_sh, edge_emb, edge_index, W1, b1, W2, b2):
    N, F = src_features.shape
    E = edge_emb.shape[0]
    EW = E // _NW
    NCH = EW // _C

    src3 = edge_index[0].reshape(_NW, NCH, _C)
    dst3 = edge_index[1].reshape(_NW, NCH, _C)

    scale = 0.25
    W1T = W1.T.astype(jnp.bfloat16)
    b1r = b1.reshape(1, F)
    W2n_s = ((W2.reshape(F, F, F).reshape(F, F * F)) * scale
             ).astype(jnp.bfloat16)
    T2 = jnp.tile(jnp.eye(F, dtype=jnp.bfloat16), (1, F))
    S2 = jnp.kron(jnp.eye(F, dtype=jnp.bfloat16),
                  jnp.ones((F, 1), dtype=jnp.bfloat16))
    b2m_s = (b2.reshape(F, F) * scale).astype(jnp.bfloat16)

    xg = _gather_call(src_features, src3, E, N, F)
    msg = _msg_call(xg, edge_sh, edge_emb, W1T, b1r, W2n_s, T2, S2, b2m_s,
                    E, F)
    pmsg, pdeg = _scatter_call(msg, dst3, E, N, F)
    return _combine_call(pmsg, pdeg, N, F)
